# baseline (speedup 1.0000x reference)
import jax
import jax.numpy as jnp
from jax import lax
from jax.experimental import pallas as pl
from jax.experimental.pallas import tpu as pltpu

N = 400000
V = 40000
C = 64
D = 16
EPS = 1e-3
B = 2000
NB = N // B

_f32 = jnp.float32


def _moments0_kernel(x_ref, mu_ref, s_ref):
    g = pl.program_id(0)

    @pl.when(g == 0)
    def _():
        mu_ref[...] = jnp.zeros((8, D), _f32)
        s_ref[...] = jnp.zeros((D, D), _f32)

    x = x_ref[...]
    mu_ref[0:1, :] += jnp.sum(x, axis=0, keepdims=True)
    s_ref[...] += lax.dot_general(x, x, (((0,), (0,)), ((), ())),
                                  preferred_element_type=_f32)


def _scan_masks(ids, d, forward):
    if forward:
        pid = jnp.concatenate(
            [jnp.full((d, 1), -1.0, _f32), ids[:-d, :]], axis=0)
    else:
        pid = jnp.concatenate(
            [ids[d:, :], jnp.full((d, 1), -1.0, _f32)], axis=0)
    return (ids == pid).astype(_f32)


def _shift(x, d, forward):
    pad = jnp.zeros((d, x.shape[1]), _f32)
    if forward:
        return jnp.concatenate([pad, x[:-d, :]], axis=0)
    return jnp.concatenate([x[d:, :], pad], axis=0)


def _segsum_kernel(forward, feats_ref, ids_ref, w_ref, sc_ref, sh_ref,
                   oh_ref, oc_ref, carry_ref):
    g = pl.program_id(0)

    @pl.when(g == 0)
    def _():
        carry_ref[...] = jnp.zeros((8, 128), _f32)
        carry_ref[2:3, 0:1] = jnp.full((1, 1), -1.0, _f32)

    ids = ids_ref[...]
    h = jnp.dot(feats_ref[...], w_ref[...], preferred_element_type=_f32)
    h = jnp.maximum(h * sc_ref[0:1, :] + sh_ref[0:1, :], 0.0)

    x = h
    c = jnp.ones((B, 1), _f32)
    d = 1
    while d < B:
        m = _scan_masks(ids, d, forward)
        x = x + _shift(x, d, forward) * m
        c = c + _shift(c, d, forward) * m
        d *= 2

    cid = carry_ref[2:3, 0:1]
    csum = carry_ref[0:1, 0:C]
    ccnt = carry_ref[1:2, 0:1]
    m0 = (ids == cid).astype(_f32)
    x = x + csum * m0
    c = c + ccnt * m0

    oh_ref[...] = x
    oc_ref[...] = c
    edge = (B - 1, B) if forward else (0, 1)
    carry_ref[0:1, 0:C] = x[edge[0]:edge[1], :]
    carry_ref[1:2, 0:1] = c[edge[0]:edge[1], :]
    carry_ref[2:3, 0:1] = ids[edge[0]:edge[1], :]


def _mid_kernel(feats_ref, fh_ref, fc_ref, bh_ref, bc_ref,
                w0_ref, sc_ref, sh_ref, w1_ref,
                y1_ref, mu_ref, s_ref):
    g = pl.program_id(0)

    @pl.when(g == 0)
    def _():
        mu_ref[...] = jnp.zeros((8, 2 * C), _f32)
        s_ref[...] = jnp.zeros((2 * C, 2 * C), _f32)

    h = jnp.dot(feats_ref[...], w0_ref[...], preferred_element_type=_f32)
    h = jnp.maximum(h * sc_ref[0:1, :] + sh_ref[0:1, :], 0.0)
    tot = fh_ref[...] + bh_ref[...] - h
    cnt = fc_ref[...] + bc_ref[...] - 1.0
    mean = tot / cnt
    f2 = jnp.concatenate([h, mean], axis=1)
    y1_ref[...] = jnp.dot(f2, w1_ref[...], preferred_element_type=_f32)
    mu_ref[0:1, :] += jnp.sum(f2, axis=0, keepdims=True)
    s_ref[...] += lax.dot_general(f2, f2, (((0,), (0,)), ((), ())),
                                  preferred_element_type=_f32)


def _segmax_kernel(y_ref, ids_ref, sc_ref, sh_ref, o_ref, carry_ref):
    g = pl.program_id(0)

    @pl.when(g == 0)
    def _():
        carry_ref[...] = jnp.zeros((8, 128), _f32)
        carry_ref[2:3, 0:1] = jnp.full((1, 1), -1.0, _f32)

    ids = ids_ref[...]
    x = jnp.maximum(y_ref[...] * sc_ref[0:1, :] + sh_ref[0:1, :], 0.0)
    d = 1
    while d < B:
        m = _scan_masks(ids, d, True)
        x = jnp.maximum(x, _shift(x, d, True) * m)
        d *= 2
    m0 = (ids == carry_ref[2:3, 0:1]).astype(_f32)
    x = jnp.maximum(x, carry_ref[0:1, 0:C] * m0)
    o_ref[...] = x
    carry_ref[0:1, 0:C] = x[B - 1:B, :]
    carry_ref[2:3, 0:1] = ids[B - 1:B, :]


def _row_spec(width, reverse=False):
    if reverse:
        return pl.BlockSpec((B, width), lambda g: (NB - 1 - g, 0))
    return pl.BlockSpec((B, width), lambda g: (g, 0))


def _full_spec(shape):
    return pl.BlockSpec(shape, lambda g: (0, 0))


def _bn_consts(mu_row, s_mat, w, g, b):
    m = (mu_row / N) @ w
    t = (s_mat / N) @ w
    e2 = jnp.sum(w * t, axis=0)
    v = e2 - m * m
    inv = g / jnp.sqrt(v + EPS)
    sc = jnp.tile(inv[None, :], (8, 1))
    sh = jnp.tile((b - m * inv)[None, :], (8, 1))
    return sc, sh


def kernel(points, f_center, f_cluster, f_relative, unq_inv,
           W0, g0, b0, W1, g1, b1):
    feats = jnp.concatenate(
        [f_center, points[:, 1:], f_cluster, f_relative], axis=-1)
    feats = jnp.pad(feats, ((0, 0), (0, D - 13)))
    w0p = jnp.pad(W0, ((0, D - 13), (0, 0)))
    idsf = unq_inv.astype(_f32).reshape(N, 1)

    mu0, s0 = pl.pallas_call(
        _moments0_kernel,
        grid=(NB,),
        in_specs=[_row_spec(D)],
        out_specs=[_full_spec((8, D)), _full_spec((D, D))],
        out_shape=[jax.ShapeDtypeStruct((8, D), _f32),
                   jax.ShapeDtypeStruct((D, D), _f32)],
    )(feats)
    sc0, sh0 = _bn_consts(mu0[0], s0, w0p, g0, b0)

    seg_args = dict(
        grid=(NB,),
        out_shape=[jax.ShapeDtypeStruct((N, C), _f32),
                   jax.ShapeDtypeStruct((N, 1), _f32)],
        scratch_shapes=[pltpu.VMEM((8, 128), _f32)],
    )
    fwd_h, fwd_c = pl.pallas_call(
        lambda *a: _segsum_kernel(True, *a),
        in_specs=[_row_spec(D), _row_spec(1), _full_spec((D, C)),
                  _full_spec((8, C)), _full_spec((8, C))],
        out_specs=[_row_spec(C), _row_spec(1)],
        **seg_args,
    )(feats, idsf, w0p, sc0, sh0)
    bwd_h, bwd_c = pl.pallas_call(
        lambda *a: _segsum_kernel(False, *a),
        in_specs=[_row_spec(D, True), _row_spec(1, True), _full_spec((D, C)),
                  _full_spec((8, C)), _full_spec((8, C))],
        out_specs=[_row_spec(C, True), _row_spec(1, True)],
        **seg_args,
    )(feats, idsf, w0p, sc0, sh0)

    y1, mu1, s1 = pl.pallas_call(
        _mid_kernel,
        grid=(NB,),
        in_specs=[_row_spec(D), _row_spec(C), _row_spec(1), _row_spec(C),
                  _row_spec(1), _full_spec((D, C)), _full_spec((8, C)),
                  _full_spec((8, C)), _full_spec((2 * C, C))],
        out_specs=[_row_spec(C), _full_spec((8, 2 * C)),
                   _full_spec((2 * C, 2 * C))],
        out_shape=[jax.ShapeDtypeStruct((N, C), _f32),
                   jax.ShapeDtypeStruct((8, 2 * C), _f32),
                   jax.ShapeDtypeStruct((2 * C, 2 * C), _f32)],
    )(feats, fwd_h, fwd_c, bwd_h, bwd_c, w0p, sc0, sh0, W1)
    sc1, sh1 = _bn_consts(mu1[0], s1, W1, g1, b1)

    fmax = pl.pallas_call(
        _segmax_kernel,
        grid=(NB,),
        in_specs=[_row_spec(C), _row_spec(1), _full_spec((8, C)),
                  _full_spec((8, C))],
        out_specs=_row_spec(C),
        out_shape=jax.ShapeDtypeStruct((N, C), _f32),
        scratch_shapes=[pltpu.VMEM((8, 128), _f32)],
    )(y1, idsf, sc1, sh1)

    vids = jnp.arange(V, dtype=unq_inv.dtype)
    pos = jnp.searchsorted(unq_inv, vids, side='right') - 1
    safe = jnp.clip(pos, 0, N - 1)
    valid = (pos >= 0) & (jnp.take(unq_inv, safe) == vids)
    return jnp.where(valid[:, None], jnp.take(fmax, safe, axis=0), 0.0)


# speedup vs baseline: 1.0808x; 1.0808x over previous
import jax
import jax.numpy as jnp
from jax import lax
from jax.experimental import pallas as pl
from jax.experimental.pallas import tpu as pltpu

N = 400000
V = 40000
C = 64
D = 16
EPS = 1e-3
B = 2000
NB = N // B

_f32 = jnp.float32


def _moments0_kernel(x_ref, mu_ref, s_ref):
    g = pl.program_id(0)

    @pl.when(g == 0)
    def _():
        mu_ref[...] = jnp.zeros((8, D), _f32)
        s_ref[...] = jnp.zeros((D, D), _f32)

    x = x_ref[...]
    mu_ref[0:1, :] += jnp.sum(x, axis=0, keepdims=True)
    s_ref[...] += lax.dot_general(x, x, (((0,), (0,)), ((), ())),
                                  preferred_element_type=_f32)


def _scan_masks(ids, d, forward):
    if forward:
        pid = jnp.concatenate(
            [jnp.full((d, 1), -1.0, _f32), ids[:-d, :]], axis=0)
    else:
        pid = jnp.concatenate(
            [ids[d:, :], jnp.full((d, 1), -1.0, _f32)], axis=0)
    return (ids == pid).astype(_f32)


def _shift(x, d, forward):
    pad = jnp.zeros((d, x.shape[1]), _f32)
    if forward:
        return jnp.concatenate([pad, x[:-d, :]], axis=0)
    return jnp.concatenate([x[d:, :], pad], axis=0)


def _segsum_kernel(forward, feats_ref, ids_ref, w_ref, sc_ref, sh_ref,
                   oh_ref, oc_ref, carry_ref):
    g = pl.program_id(0)

    @pl.when(g == 0)
    def _():
        carry_ref[...] = jnp.zeros((8, 128), _f32)
        carry_ref[2:3, 0:1] = jnp.full((1, 1), -1.0, _f32)

    ids = ids_ref[...]
    h = jnp.dot(feats_ref[...], w_ref[...], preferred_element_type=_f32)
    h = jnp.maximum(h * sc_ref[0:1, :] + sh_ref[0:1, :], 0.0)

    x = jnp.concatenate(
        [h, jnp.ones((B, 1), _f32), jnp.zeros((B, 127 - C), _f32)], axis=1)
    d = 1
    while d < B:
        m = _scan_masks(ids, d, forward)
        x = x + _shift(x, d, forward) * m
        d *= 2

    m0 = (ids == carry_ref[2:3, 0:1]).astype(_f32)
    x = x + carry_ref[0:1, :] * m0

    oh_ref[...] = x[:, 0:C]
    oc_ref[...] = x[:, C:C + 1]
    edge = (B - 1, B) if forward else (0, 1)
    carry_ref[0:1, :] = x[edge[0]:edge[1], :]
    carry_ref[2:3, 0:1] = ids[edge[0]:edge[1], :]


def _mid_kernel(feats_ref, fh_ref, fc_ref, bh_ref, bc_ref,
                w0_ref, sc_ref, sh_ref, w1_ref,
                y1_ref, mu_ref, s_ref):
    g = pl.program_id(0)

    @pl.when(g == 0)
    def _():
        mu_ref[...] = jnp.zeros((8, 2 * C), _f32)
        s_ref[...] = jnp.zeros((2 * C, 2 * C), _f32)

    h = jnp.dot(feats_ref[...], w0_ref[...], preferred_element_type=_f32)
    h = jnp.maximum(h * sc_ref[0:1, :] + sh_ref[0:1, :], 0.0)
    tot = fh_ref[...] + bh_ref[...] - h
    cnt = fc_ref[...] + bc_ref[...] - 1.0
    mean = tot / cnt
    f2 = jnp.concatenate([h, mean], axis=1)
    y1_ref[...] = jnp.dot(f2, w1_ref[...], preferred_element_type=_f32)
    mu_ref[0:1, :] += jnp.sum(f2, axis=0, keepdims=True)
    s_ref[...] += lax.dot_general(f2, f2, (((0,), (0,)), ((), ())),
                                  preferred_element_type=_f32)


def _segmax_kernel(y_ref, ids_ref, sc_ref, sh_ref, o_ref, carry_ref):
    g = pl.program_id(0)

    @pl.when(g == 0)
    def _():
        carry_ref[...] = jnp.zeros((8, 128), _f32)
        carry_ref[2:3, 0:1] = jnp.full((1, 1), -1.0, _f32)

    ids = ids_ref[...]
    x = jnp.maximum(y_ref[...] * sc_ref[0:1, :] + sh_ref[0:1, :], 0.0)
    d = 1
    while d < B:
        m = _scan_masks(ids, d, True)
        x = jnp.maximum(x, _shift(x, d, True) * m)
        d *= 2
    m0 = (ids == carry_ref[2:3, 0:1]).astype(_f32)
    x = jnp.maximum(x, carry_ref[0:1, 0:C] * m0)
    o_ref[...] = x
    carry_ref[0:1, 0:C] = x[B - 1:B, :]
    carry_ref[2:3, 0:1] = ids[B - 1:B, :]


def _row_spec(width, reverse=False):
    if reverse:
        return pl.BlockSpec((B, width), lambda g: (NB - 1 - g, 0))
    return pl.BlockSpec((B, width), lambda g: (g, 0))


def _full_spec(shape):
    return pl.BlockSpec(shape, lambda g: (0, 0))


def _bn_consts(mu_row, s_mat, w, g, b):
    m = (mu_row / N) @ w
    t = (s_mat / N) @ w
    e2 = jnp.sum(w * t, axis=0)
    v = e2 - m * m
    inv = g / jnp.sqrt(v + EPS)
    sc = jnp.tile(inv[None, :], (8, 1))
    sh = jnp.tile((b - m * inv)[None, :], (8, 1))
    return sc, sh


def kernel(points, f_center, f_cluster, f_relative, unq_inv,
           W0, g0, b0, W1, g1, b1):
    feats = jnp.concatenate(
        [f_center, points[:, 1:], f_cluster, f_relative], axis=-1)
    feats = jnp.pad(feats, ((0, 0), (0, D - 13)))
    w0p = jnp.pad(W0, ((0, D - 13), (0, 0)))
    idsf = unq_inv.astype(_f32).reshape(N, 1)

    mu0, s0 = pl.pallas_call(
        _moments0_kernel,
        grid=(NB,),
        in_specs=[_row_spec(D)],
        out_specs=[_full_spec((8, D)), _full_spec((D, D))],
        out_shape=[jax.ShapeDtypeStruct((8, D), _f32),
                   jax.ShapeDtypeStruct((D, D), _f32)],
    )(feats)
    sc0, sh0 = _bn_consts(mu0[0], s0, w0p, g0, b0)

    seg_args = dict(
        grid=(NB,),
        out_shape=[jax.ShapeDtypeStruct((N, C), _f32),
                   jax.ShapeDtypeStruct((N, 1), _f32)],
        scratch_shapes=[pltpu.VMEM((8, 128), _f32)],
    )
    fwd_h, fwd_c = pl.pallas_call(
        lambda *a: _segsum_kernel(True, *a),
        in_specs=[_row_spec(D), _row_spec(1), _full_spec((D, C)),
                  _full_spec((8, C)), _full_spec((8, C))],
        out_specs=[_row_spec(C), _row_spec(1)],
        **seg_args,
    )(feats, idsf, w0p, sc0, sh0)
    bwd_h, bwd_c = pl.pallas_call(
        lambda *a: _segsum_kernel(False, *a),
        in_specs=[_row_spec(D, True), _row_spec(1, True), _full_spec((D, C)),
                  _full_spec((8, C)), _full_spec((8, C))],
        out_specs=[_row_spec(C, True), _row_spec(1, True)],
        **seg_args,
    )(feats, idsf, w0p, sc0, sh0)

    y1, mu1, s1 = pl.pallas_call(
        _mid_kernel,
        grid=(NB,),
        in_specs=[_row_spec(D), _row_spec(C), _row_spec(1), _row_spec(C),
                  _row_spec(1), _full_spec((D, C)), _full_spec((8, C)),
                  _full_spec((8, C)), _full_spec((2 * C, C))],
        out_specs=[_row_spec(C), _full_spec((8, 2 * C)),
                   _full_spec((2 * C, 2 * C))],
        out_shape=[jax.ShapeDtypeStruct((N, C), _f32),
                   jax.ShapeDtypeStruct((8, 2 * C), _f32),
                   jax.ShapeDtypeStruct((2 * C, 2 * C), _f32)],
    )(feats, fwd_h, fwd_c, bwd_h, bwd_c, w0p, sc0, sh0, W1)
    sc1, sh1 = _bn_consts(mu1[0], s1, W1, g1, b1)

    fmax = pl.pallas_call(
        _segmax_kernel,
        grid=(NB,),
        in_specs=[_row_spec(C), _row_spec(1), _full_spec((8, C)),
                  _full_spec((8, C))],
        out_specs=_row_spec(C),
        out_shape=jax.ShapeDtypeStruct((N, C), _f32),
        scratch_shapes=[pltpu.VMEM((8, 128), _f32)],
    )(y1, idsf, sc1, sh1)

    vids = jnp.arange(V, dtype=unq_inv.dtype)
    pos = jnp.searchsorted(unq_inv, vids, side='right') - 1
    safe = jnp.clip(pos, 0, N - 1)
    valid = (pos >= 0) & (jnp.take(unq_inv, safe) == vids)
    return jnp.where(valid[:, None], jnp.take(fmax, safe, axis=0), 0.0)


# speedup vs baseline: 1.0961x; 1.0141x over previous
import jax
import jax.numpy as jnp
from jax import lax
from jax.experimental import pallas as pl
from jax.experimental.pallas import tpu as pltpu

N = 400000
V = 40000
C = 64
D = 16
EPS = 1e-3
B = 4000
NB = N // B

_f32 = jnp.float32


def _moments0_kernel(x_ref, mu_ref, s_ref):
    g = pl.program_id(0)

    @pl.when(g == 0)
    def _():
        mu_ref[...] = jnp.zeros((8, D), _f32)
        s_ref[...] = jnp.zeros((D, D), _f32)

    x = x_ref[...]
    mu_ref[0:1, :] += jnp.sum(x, axis=0, keepdims=True)
    s_ref[...] += lax.dot_general(x, x, (((0,), (0,)), ((), ())),
                                  preferred_element_type=_f32)


def _scan_masks(ids, d, forward):
    if forward:
        pid = jnp.concatenate(
            [jnp.full((d, 1), -1.0, _f32), ids[:-d, :]], axis=0)
    else:
        pid = jnp.concatenate(
            [ids[d:, :], jnp.full((d, 1), -1.0, _f32)], axis=0)
    return (ids == pid).astype(_f32)


def _shift(x, d, forward):
    pad = jnp.zeros((d, x.shape[1]), _f32)
    if forward:
        return jnp.concatenate([pad, x[:-d, :]], axis=0)
    return jnp.concatenate([x[d:, :], pad], axis=0)


def _segsum_kernel(forward, feats_ref, ids_ref, w_ref, sc_ref, sh_ref,
                   oh_ref, oc_ref, carry_ref):
    g = pl.program_id(0)

    @pl.when(g == 0)
    def _():
        carry_ref[...] = jnp.zeros((8, 128), _f32)
        carry_ref[2:3, 0:1] = jnp.full((1, 1), -1.0, _f32)

    ids = ids_ref[...]
    h = jnp.dot(feats_ref[...], w_ref[...], preferred_element_type=_f32)
    h = jnp.maximum(h * sc_ref[0:1, :] + sh_ref[0:1, :], 0.0)

    x = jnp.concatenate(
        [h, jnp.ones((B, 1), _f32), jnp.zeros((B, 127 - C), _f32)], axis=1)
    d = 1
    while d < B:
        m = _scan_masks(ids, d, forward)
        x = x + _shift(x, d, forward) * m
        d *= 2

    m0 = (ids == carry_ref[2:3, 0:1]).astype(_f32)
    x = x + carry_ref[0:1, :] * m0

    oh_ref[...] = x[:, 0:C]
    oc_ref[...] = x[:, C:C + 1]
    edge = (B - 1, B) if forward else (0, 1)
    carry_ref[0:1, :] = x[edge[0]:edge[1], :]
    carry_ref[2:3, 0:1] = ids[edge[0]:edge[1], :]


def _mid_kernel(feats_ref, fh_ref, fc_ref, bh_ref, bc_ref,
                w0_ref, sc_ref, sh_ref, w1_ref,
                y1_ref, mu_ref, s_ref):
    g = pl.program_id(0)

    @pl.when(g == 0)
    def _():
        mu_ref[...] = jnp.zeros((8, 2 * C), _f32)
        s_ref[...] = jnp.zeros((2 * C, 2 * C), _f32)

    h = jnp.dot(feats_ref[...], w0_ref[...], preferred_element_type=_f32)
    h = jnp.maximum(h * sc_ref[0:1, :] + sh_ref[0:1, :], 0.0)
    tot = fh_ref[...] + bh_ref[...] - h
    cnt = fc_ref[...] + bc_ref[...] - 1.0
    mean = tot / cnt
    f2 = jnp.concatenate([h, mean], axis=1)
    y1_ref[...] = jnp.dot(f2, w1_ref[...], preferred_element_type=_f32)
    mu_ref[0:1, :] += jnp.sum(f2, axis=0, keepdims=True)
    s_ref[...] += lax.dot_general(f2, f2, (((0,), (0,)), ((), ())),
                                  preferred_element_type=_f32)


def _segmax_kernel(y_ref, ids_ref, sc_ref, sh_ref, o_ref, carry_ref):
    g = pl.program_id(0)

    @pl.when(g == 0)
    def _():
        carry_ref[...] = jnp.zeros((8, 128), _f32)
        carry_ref[2:3, 0:1] = jnp.full((1, 1), -1.0, _f32)

    ids = ids_ref[...]
    x = jnp.maximum(y_ref[...] * sc_ref[0:1, :] + sh_ref[0:1, :], 0.0)
    d = 1
    while d < B:
        m = _scan_masks(ids, d, True)
        x = jnp.maximum(x, _shift(x, d, True) * m)
        d *= 2
    m0 = (ids == carry_ref[2:3, 0:1]).astype(_f32)
    x = jnp.maximum(x, carry_ref[0:1, 0:C] * m0)
    o_ref[...] = x
    carry_ref[0:1, 0:C] = x[B - 1:B, :]
    carry_ref[2:3, 0:1] = ids[B - 1:B, :]


def _row_spec(width, reverse=False):
    if reverse:
        return pl.BlockSpec((B, width), lambda g: (NB - 1 - g, 0))
    return pl.BlockSpec((B, width), lambda g: (g, 0))


def _full_spec(shape):
    return pl.BlockSpec(shape, lambda g: (0, 0))


def _bn_consts(mu_row, s_mat, w, g, b):
    m = (mu_row / N) @ w
    t = (s_mat / N) @ w
    e2 = jnp.sum(w * t, axis=0)
    v = e2 - m * m
    inv = g / jnp.sqrt(v + EPS)
    sc = jnp.tile(inv[None, :], (8, 1))
    sh = jnp.tile((b - m * inv)[None, :], (8, 1))
    return sc, sh


def kernel(points, f_center, f_cluster, f_relative, unq_inv,
           W0, g0, b0, W1, g1, b1):
    feats = jnp.concatenate(
        [f_center, points[:, 1:], f_cluster, f_relative], axis=-1)
    feats = jnp.pad(feats, ((0, 0), (0, D - 13)))
    w0p = jnp.pad(W0, ((0, D - 13), (0, 0)))
    idsf = unq_inv.astype(_f32).reshape(N, 1)

    mu0, s0 = pl.pallas_call(
        _moments0_kernel,
        grid=(NB,),
        in_specs=[_row_spec(D)],
        out_specs=[_full_spec((8, D)), _full_spec((D, D))],
        out_shape=[jax.ShapeDtypeStruct((8, D), _f32),
                   jax.ShapeDtypeStruct((D, D), _f32)],
    )(feats)
    sc0, sh0 = _bn_consts(mu0[0], s0, w0p, g0, b0)

    seg_args = dict(
        grid=(NB,),
        out_shape=[jax.ShapeDtypeStruct((N, C), _f32),
                   jax.ShapeDtypeStruct((N, 1), _f32)],
        scratch_shapes=[pltpu.VMEM((8, 128), _f32)],
    )
    fwd_h, fwd_c = pl.pallas_call(
        lambda *a: _segsum_kernel(True, *a),
        in_specs=[_row_spec(D), _row_spec(1), _full_spec((D, C)),
                  _full_spec((8, C)), _full_spec((8, C))],
        out_specs=[_row_spec(C), _row_spec(1)],
        **seg_args,
    )(feats, idsf, w0p, sc0, sh0)
    bwd_h, bwd_c = pl.pallas_call(
        lambda *a: _segsum_kernel(False, *a),
        in_specs=[_row_spec(D, True), _row_spec(1, True), _full_spec((D, C)),
                  _full_spec((8, C)), _full_spec((8, C))],
        out_specs=[_row_spec(C, True), _row_spec(1, True)],
        **seg_args,
    )(feats, idsf, w0p, sc0, sh0)

    y1, mu1, s1 = pl.pallas_call(
        _mid_kernel,
        grid=(NB,),
        in_specs=[_row_spec(D), _row_spec(C), _row_spec(1), _row_spec(C),
                  _row_spec(1), _full_spec((D, C)), _full_spec((8, C)),
                  _full_spec((8, C)), _full_spec((2 * C, C))],
        out_specs=[_row_spec(C), _full_spec((8, 2 * C)),
                   _full_spec((2 * C, 2 * C))],
        out_shape=[jax.ShapeDtypeStruct((N, C), _f32),
                   jax.ShapeDtypeStruct((8, 2 * C), _f32),
                   jax.ShapeDtypeStruct((2 * C, 2 * C), _f32)],
    )(feats, fwd_h, fwd_c, bwd_h, bwd_c, w0p, sc0, sh0, W1)
    sc1, sh1 = _bn_consts(mu1[0], s1, W1, g1, b1)

    fmax = pl.pallas_call(
        _segmax_kernel,
        grid=(NB,),
        in_specs=[_row_spec(C), _row_spec(1), _full_spec((8, C)),
                  _full_spec((8, C))],
        out_specs=_row_spec(C),
        out_shape=jax.ShapeDtypeStruct((N, C), _f32),
        scratch_shapes=[pltpu.VMEM((8, 128), _f32)],
    )(y1, idsf, sc1, sh1)

    vids = jnp.arange(V, dtype=unq_inv.dtype)
    pos = jnp.searchsorted(unq_inv, vids, side='right') - 1
    safe = jnp.clip(pos, 0, N - 1)
    valid = (pos >= 0) & (jnp.take(unq_inv, safe) == vids)
    return jnp.where(valid[:, None], jnp.take(fmax, safe, axis=0), 0.0)
